# baseline (speedup 1.0000x reference)
import functools
import math

import jax
from jax import lax
import jax.numpy as jnp
from jax.experimental import pallas as pl
from jax.experimental.pallas import tpu as pltpu

_S = 64.0
_RB = 8
_NBUF = 16
_LOOK = 8
_NSEM = 5


def _acos(x):
    p = jnp.float32(-0.0012624911)
    p = p * x + jnp.float32(0.0066700901)
    p = p * x - jnp.float32(0.0170881256)
    p = p * x + jnp.float32(0.0308918810)
    p = p * x - jnp.float32(0.0501743046)
    p = p * x + jnp.float32(0.0889789874)
    p = p * x - jnp.float32(0.2145988016)
    p = p * x + jnp.float32(1.5707963050)
    return jnp.sqrt(1.0 - x) * p


def _main_body(lab_ref, x_hbm, out_hbm, tgt_ref, win_ref, sum_ref,
               *rest):
    bufs = list(rest[:_NBUF])
    acc_ref, gsem, ssem = rest[_NBUF:]
    b, n = x_hbm.shape
    nstripes = b // _RB

    def gather(t, bi):
        row0 = pl.multiple_of(t * _RB, _RB)
        return pltpu.make_async_copy(
            x_hbm.at[pl.ds(row0, _RB), :], bufs[bi],
            gsem.at[jax.lax.rem(t, _NSEM)])

    def scatter(t, bi):
        row0 = pl.multiple_of(t * _RB, _RB)
        return pltpu.make_async_copy(
            bufs[bi], out_hbm.at[pl.ds(row0, _RB), :],
            ssem.at[jax.lax.rem(t, _NSEM)])

    acc_ref[0] = 0.0
    for t0 in range(_LOOK):
        gather(t0, t0).start()

    def outer(g, carry):
        for bi in range(_NBUF):
            t = g * _NBUF + bi

            @pl.when(t >= _NBUF - _LOOK)
            def _():
                scatter(t - (_NBUF - _LOOK), (bi + _LOOK) % _NBUF).wait()

            @pl.when(t + _LOOK < nstripes)
            def _():
                gather(t + _LOOK, (bi + _LOOK) % _NBUF).start()

            gather(t, bi).wait()

            buf = bufs[bi]
            x = buf[...]
            acc_ref[0] += jnp.sum(x)

            tiles = []
            lanes = []
            for r in range(_RB):
                col = lab_ref[t * _RB + r]
                col_tile = pl.multiple_of((col // 128) * 128, 128)
                tiles.append(buf[pl.ds(r, 1), pl.ds(col_tile, 128)])
                lanes.append(col - col_tile)
            win = jnp.concatenate(tiles, axis=0)
            row0 = pl.multiple_of(t * _RB, _RB)
            win_ref[pl.ds(row0, _RB), :] = win * _S
            lane = jnp.concatenate(
                [jnp.full((1, 1), l, jnp.int32) for l in lanes], axis=0)
            lane_iota = jax.lax.broadcasted_iota(jnp.int32, (_RB, 128), 1)
            picked = jnp.where(lane_iota == lane, win, 0.0)
            tgt_ref[pl.ds(row0, _RB), :] = jnp.sum(picked, axis=1, keepdims=True)

            buf[...] = x * _S
            scatter(t, bi).start()
        return carry

    lax.fori_loop(0, nstripes // _NBUF, outer, 0)
    sum_ref[0, 0] = acc_ref[0]
    for td in range(nstripes - (_NBUF - _LOOK), nstripes):
        scatter(td, td % _NBUF).wait()


def _fixup_body(out_in, tgt_ref, win_ref, lab2_ref, tot_ref, lab_ref, out_hbm,
                blend_ref, sem):
    del out_in
    b, n = out_hbm.shape
    t = tgt_ref[...]
    pos_sum = jnp.sum(t)
    total = tot_ref[0, 0]
    pos_mean = pos_sum / b
    neg_mean = (total - pos_sum) / (b * (n - 1))
    avg_p_theta = _acos(pos_mean)
    c = jnp.float32(math.log(n - 1) / _S)
    d_m = jnp.log(_acos(neg_mean + c) / math.pi) / jnp.log(avg_p_theta / math.pi)
    theta = _acos(t)
    ratio = theta * jnp.float32(1.0 / math.pi)
    warped = jnp.exp(d_m * jnp.log(ratio)) * jnp.float32(math.pi)
    final = jnp.cos(warped) * _S

    lane = jax.lax.rem(lab2_ref[...], jnp.int32(128))
    lane_iota = jax.lax.broadcasted_iota(jnp.int32, (b, 128), 1)
    blend_ref[...] = jnp.where(lane_iota == lane, final, win_ref[...])

    def _copy(i):
        col = lab_ref[i]
        col_tile = pl.multiple_of((col // 128) * 128, 128)
        return pltpu.make_async_copy(
            blend_ref.at[pl.ds(i, 1), :],
            out_hbm.at[pl.ds(i, 1), pl.ds(col_tile, 128)],
            sem,
        )

    def _start(i, _):
        _copy(i).start()
        return 0

    def _wait(i, _):
        _copy(i).wait()
        return 0

    jax.lax.fori_loop(0, b, _start, 0)
    jax.lax.fori_loop(0, b, _wait, 0)


@jax.jit
def kernel(logits, labels):
    b, n = logits.shape

    out0, tgt, wins, total = pl.pallas_call(
        _main_body,
        in_specs=[
            pl.BlockSpec(memory_space=pltpu.SMEM),
            pl.BlockSpec(memory_space=pl.ANY),
        ],
        out_specs=[
            pl.BlockSpec(memory_space=pl.ANY),
            pl.BlockSpec(memory_space=pltpu.VMEM),
            pl.BlockSpec(memory_space=pltpu.VMEM),
            pl.BlockSpec(memory_space=pltpu.SMEM),
        ],
        out_shape=[
            jax.ShapeDtypeStruct((b, n), jnp.float32),
            jax.ShapeDtypeStruct((b, 1), jnp.float32),
            jax.ShapeDtypeStruct((b, 128), jnp.float32),
            jax.ShapeDtypeStruct((1, 1), jnp.float32),
        ],
        scratch_shapes=[pltpu.VMEM((_RB, n), jnp.float32)] * _NBUF
        + [pltpu.SMEM((1,), jnp.float32),
           pltpu.SemaphoreType.DMA((_NSEM,)), pltpu.SemaphoreType.DMA((_NSEM,))],
        compiler_params=pltpu.CompilerParams(
            vmem_limit_bytes=100 * 1024 * 1024),
    )(labels, logits)

    lab2 = labels.reshape(b, 1)

    out = pl.pallas_call(
        _fixup_body,
        in_specs=[
            pl.BlockSpec(memory_space=pl.ANY),
            pl.BlockSpec(memory_space=pltpu.VMEM),
            pl.BlockSpec(memory_space=pltpu.VMEM),
            pl.BlockSpec(memory_space=pltpu.VMEM),
            pl.BlockSpec(memory_space=pltpu.SMEM),
            pl.BlockSpec(memory_space=pltpu.SMEM),
        ],
        out_specs=pl.BlockSpec(memory_space=pl.ANY),
        out_shape=jax.ShapeDtypeStruct((b, n), jnp.float32),
        input_output_aliases={0: 0},
        scratch_shapes=[
            pltpu.VMEM((b, 128), jnp.float32),
            pltpu.SemaphoreType.DMA,
        ],
    )(out0, tgt, wins, lab2, total, labels)
    return out
